# baseline (speedup 1.0000x reference)
import functools

import jax
import jax.numpy as jnp
from jax import lax
from jax.experimental import pallas as pl
from jax.experimental.pallas import tpu as pltpu
from jax.experimental.pallas import tpu_sc as plsc

F32 = jnp.float32

N = 10000
E = 160000
NC, NS = 2, 16
NW = NC * NS
EP = 5120
CH = 128
NCHUNK = EP // CH
PAD_E = NW * EP
NQ = 3456
QP = 3
NPAD = QP * NQ
ACC_R = 3584
ZSLAB = ACC_R // NS
FSLAB = NQ // NS
TRASH = 3576
COLPAD = 10200


def _mesh():
    return plsc.VectorSubcoreMesh(
        core_axis_name="c", subcore_axis_name="s", num_cores=NC, num_subcores=NS)


@functools.lru_cache(maxsize=None)
def _make_gather():
    @functools.partial(
        pl.kernel,
        mesh=_mesh(),
        out_type=(
            jax.ShapeDtypeStruct((PAD_E, 128), F32),
            jax.ShapeDtypeStruct((PAD_E, 128), F32),
        ),
        scratch_types=[
            pltpu.VMEM((8 * CH,), jnp.int32),
            pltpu.VMEM((8 * CH,), jnp.int32),
            pltpu.VMEM((2, CH, 128), F32),
            pltpu.VMEM((2, CH, 128), F32),
            pltpu.SemaphoreType.DMA,
            pltpu.SemaphoreType.DMA,
            pltpu.SemaphoreType.DMA,
            pltpu.SemaphoreType.DMA,
        ],
    )
    def gather(ta, ia, tb, ib, oa, ob, iva, ivb, bufa, bufb,
               sga, sgb, swa, swb):
        wid = lax.axis_index("s") * NC + lax.axis_index("c")
        base = wid * EP

        SLABC = 8
        for s in range(NCHUNK // SLABC):
            j0 = s * SLABC
            pltpu.sync_copy(ia.at[pl.ds(base + j0 * CH, SLABC * CH)], iva)
            pltpu.sync_copy(ib.at[pl.ds(base + j0 * CH, SLABC * CH)], ivb)

            @pl.loop(0, SLABC, step=2)
            def step(jj):
                for b in (0, 1):
                    lc = jj + b
                    j = j0 + lc
                    off = base + j * CH

                    @pl.when(j >= 2)
                    def _():
                        pltpu.make_async_copy(
                            bufa.at[b], oa.at[pl.ds(off, CH)], swa).wait()
                        pltpu.make_async_copy(
                            bufb.at[b], ob.at[pl.ds(off, CH)], swb).wait()

                    ca = pltpu.async_copy(
                        ta.at[iva.at[pl.ds(lc * CH, CH)]], bufa.at[b], sga)
                    cb = pltpu.async_copy(
                        tb.at[ivb.at[pl.ds(lc * CH, CH)]], bufb.at[b], sgb)
                    ca.wait()
                    cb.wait()
                    pltpu.async_copy(bufa.at[b], oa.at[pl.ds(off, CH)], swa)
                    pltpu.async_copy(bufb.at[b], ob.at[pl.ds(off, CH)], swb)

        for b in (0, 1):
            pltpu.make_async_copy(bufa.at[b], oa.at[pl.ds(base, CH)], swa).wait()
            pltpu.make_async_copy(bufb.at[b], ob.at[pl.ds(base, CH)], swb).wait()

    return gather


@functools.lru_cache(maxsize=None)
def _make_scatter():
    @functools.partial(
        pl.kernel,
        mesh=_mesh(),
        out_type=jax.ShapeDtypeStruct((NC * NPAD, 128), F32),
        scratch_types=[
            pltpu.VMEM((8, CH), jnp.int32),
            pltpu.VMEM((2, CH, 128), F32),
            pltpu.VMEM((FSLAB, 128), F32),
            pltpu.VMEM_SHARED((ACC_R, 128), F32),
            pltpu.SemaphoreType.DMA,
        ],
    )
    def scat(rows, cq2, zeros, out, islab, rbuf, stage, acc, sadd):
        cid = lax.axis_index("c")
        sid = lax.axis_index("s")
        wid = sid * NC + cid
        base = wid * EP

        for p in range(QP):
            pltpu.sync_copy(zeros, acc.at[pl.ds(sid * ZSLAB, ZSLAB)])
            plsc.subcore_barrier()

            for s in range(NCHUNK // 8):
                if s > 0:
                    for b in (0, 1):
                        pltpu.make_async_copy(
                            rbuf.at[b], acc.at[islab.at[b]], sadd).wait()
                pltpu.sync_copy(
                    cq2.at[pl.ds(p * (PAD_E // CH) + wid * NCHUNK + s * 8, 8)],
                    islab)

                @pl.loop(0, 8, step=2)
                def step(jj):
                    for b in (0, 1):
                        lc = jj + b
                        off = base + (s * 8) * CH + lc * CH

                        @pl.when(lc >= 2)
                        def _():
                            pltpu.make_async_copy(
                                rbuf.at[b], acc.at[islab.at[b]], sadd).wait()

                        pltpu.sync_copy(rows.at[pl.ds(off, CH)], rbuf.at[b])
                        pltpu.async_copy(rbuf.at[b], acc.at[islab.at[lc]],
                                         sadd, add=True)

            for b in (0, 1):
                pltpu.make_async_copy(rbuf.at[b], acc.at[islab.at[b]],
                                      sadd).wait()
            plsc.subcore_barrier()
            pltpu.sync_copy(acc.at[pl.ds(sid * FSLAB, FSLAB)], stage)
            pltpu.sync_copy(
                stage,
                out.at[pl.ds(cid * NPAD + p * NQ + sid * FSLAB, FSLAB)])
            plsc.subcore_barrier()

    return scat


@functools.lru_cache(maxsize=None)
def _make_counts():
    @functools.partial(
        pl.kernel,
        mesh=_mesh(),
        compiler_params=pltpu.CompilerParams(needs_layout_passes=False),
        out_type=jax.ShapeDtypeStruct((NW * NPAD,), F32),
        scratch_types=[
            pltpu.VMEM((1024,), jnp.int32),
            pltpu.VMEM((NPAD,), F32),
        ],
    )
    def cnt(cols, out, iv, acc):
        cid = lax.axis_index("c")
        sid = lax.axis_index("s")
        wid = sid * NC + cid
        base = wid * EP
        zero = jnp.zeros((16,), F32)
        ones = jnp.ones((16,), F32)

        def zloop(i, carry):
            acc[pl.ds(i * 16, 16)] = zero
            return carry

        lax.fori_loop(0, NPAD // 16, zloop, 0)

        def body(j, carry):
            pltpu.sync_copy(cols.at[pl.ds(base + j * 1024, 1024)], iv)

            def inner(k, c2):
                plsc.addupdate_scatter(acc, [iv[pl.ds(k * 16, 16)]], ones)
                return c2

            lax.fori_loop(0, 64, inner, 0)
            return carry

        lax.fori_loop(0, EP // 1024, body, 0)
        pltpu.sync_copy(acc, out.at[pl.ds(wid * NPAD, NPAD)])

    return cnt


BE = 2048
GE = PAD_E // BE
BN = 1000
GN = N // BN


def _dot(a, b):
    return jnp.dot(a, b, preferred_element_type=F32)


def _pre_body(x_ref, wu_ref, wv_ref, u_ref, v_ref):
    x = x_ref[...]
    u_ref[...] = _dot(x, wu_ref[...])
    v_ref[...] = _dot(x, wv_ref[...])


def _pre_l1(x8, wu, wv):
    return pl.pallas_call(
        _pre_body,
        grid=(GN,),
        in_specs=[
            pl.BlockSpec((BN, 8), lambda i: (i, 0)),
            pl.BlockSpec((8, 128), lambda i: (0, 0)),
            pl.BlockSpec((8, 128), lambda i: (0, 0)),
        ],
        out_specs=[
            pl.BlockSpec((BN, 128), lambda i: (i, 0)),
            pl.BlockSpec((BN, 128), lambda i: (i, 0)),
        ],
        out_shape=[
            jax.ShapeDtypeStruct((N, 128), F32),
            jax.ShapeDtypeStruct((N, 128), F32),
        ],
    )(x8, wu, wv)


def _edge_body(us_ref, vd_ref, ea_ref, we_ref, w_ref, b_ref, eh_ref):
    u = us_ref[...]
    b = b_ref[...]
    t = (u[:, :64] + vd_ref[...][:, :64]
         + _dot(ea_ref[...][:, 64:], we_ref[...]) + b[0:1])
    t = jnp.maximum(t, 0.0)
    ea2 = _dot(t, w_ref[0]) + b[1:2]
    z = u[:, 64:] + _dot(ea2, w_ref[1]) + b[2:3]
    z = jnp.maximum(z, 0.0)
    h = _dot(z, w_ref[2]) + b[3:4]
    eh_ref[...] = jnp.concatenate([h, ea2], axis=1)


def _edge(us, vd, ea, we, w, b):
    return pl.pallas_call(
        _edge_body,
        grid=(GE,),
        in_specs=[
            pl.BlockSpec((BE, 128), lambda i: (i, 0)),
            pl.BlockSpec((BE, 128), lambda i: (i, 0)),
            pl.BlockSpec((BE, 128), lambda i: (i, 0)),
            pl.BlockSpec((64, 64), lambda i: (0, 0)),
            pl.BlockSpec((3, 64, 64), lambda i: (0, 0, 0)),
            pl.BlockSpec((8, 64), lambda i: (0, 0)),
        ],
        out_specs=pl.BlockSpec((BE, 128), lambda i: (i, 0)),
        out_shape=jax.ShapeDtypeStruct((PAD_E, 128), F32),
    )(us, vd, ea, we, w, b)


def _edge5_body(us_ref, vd_ref, ea_ref, we_ref, w2_ref, b_ref, o_ref):
    u = us_ref[...]
    b = b_ref[...]
    t = (u[:, :64] + vd_ref[...][:, :64]
         + _dot(ea_ref[...][:, 64:], we_ref[...]) + b[0:1])
    t = jnp.maximum(t, 0.0)
    o = _dot(t, w2_ref[...]) + b[1:2, :8]
    o_ref[...] = jax.nn.sigmoid(o)


def _edge_l5(us, vd, ea, we, w2, b):
    return pl.pallas_call(
        _edge5_body,
        grid=(GE,),
        in_specs=[
            pl.BlockSpec((BE, 128), lambda i: (i, 0)),
            pl.BlockSpec((BE, 128), lambda i: (i, 0)),
            pl.BlockSpec((BE, 128), lambda i: (i, 0)),
            pl.BlockSpec((64, 64), lambda i: (0, 0)),
            pl.BlockSpec((64, 8), lambda i: (0, 0)),
            pl.BlockSpec((8, 64), lambda i: (0, 0)),
        ],
        out_specs=pl.BlockSpec((BE, 8), lambda i: (i, 0)),
        out_shape=jax.ShapeDtypeStruct((PAD_E, 8), F32),
    )(us, vd, ea, we, w2, b)


def _node_body(x_ref, p_ref, cp_ref, wx_ref, w_ref, wts_ref, wtd_ref, b_ref,
               xo_ref, ts_ref, td_ref):
    p = p_ref[...]
    cp = cp_ref[...]
    b = b_ref[...]
    s = p[0, :, :64] + p[1, :, :64]
    c = jnp.sum(cp, axis=1, keepdims=True)
    agg = s / jnp.maximum(c, 1.0)
    t = _dot(x_ref[...], wx_ref[...]) + _dot(agg, w_ref[0]) + b[0:1]
    t = jnp.maximum(t, 0.0)
    xo = _dot(t, w_ref[1]) + b[1:2]
    xo_ref[...] = xo
    ts_ref[...] = _dot(xo, wts_ref[...])
    td_ref[...] = _dot(xo, wtd_ref[...])


def _node(x, part, cntp, wx, w, wts, wtd, b):
    return pl.pallas_call(
        _node_body,
        grid=(GN,),
        in_specs=[
            pl.BlockSpec((BN, 64), lambda i: (i, 0)),
            pl.BlockSpec((NC, BN, 128), lambda i: (0, i, 0)),
            pl.BlockSpec((BN, NW), lambda i: (i, 0)),
            pl.BlockSpec((64, 64), lambda i: (0, 0)),
            pl.BlockSpec((2, 64, 64), lambda i: (0, 0, 0)),
            pl.BlockSpec((64, 128), lambda i: (0, 0)),
            pl.BlockSpec((64, 128), lambda i: (0, 0)),
            pl.BlockSpec((8, 64), lambda i: (0, 0)),
        ],
        out_specs=[
            pl.BlockSpec((BN, 64), lambda i: (i, 0)),
            pl.BlockSpec((BN, 128), lambda i: (i, 0)),
            pl.BlockSpec((BN, 128), lambda i: (i, 0)),
        ],
        out_shape=[
            jax.ShapeDtypeStruct((N, 64), F32),
            jax.ShapeDtypeStruct((N, 128), F32),
            jax.ShapeDtypeStruct((N, 128), F32),
        ],
    )(x, part, cntp, wx, w, wts, wtd, b)


def _pad_rows(a, rows):
    return jnp.concatenate(
        [a, jnp.zeros((rows - a.shape[0],) + a.shape[1:], a.dtype)], axis=0)


def _pad_cols(a, cols):
    return jnp.pad(a, ((0, 0), (0, cols - a.shape[1])))


def _bstack(*bs):
    out = jnp.zeros((8, 64), F32)
    for i, b in enumerate(bs):
        out = out.at[i, : b.shape[0]].set(b)
    return out


def kernel(x, edge_index, edge_attr, params):
    row, col = edge_index[0], edge_index[1]
    padn = PAD_E - E
    rowp = jnp.concatenate([row, jnp.zeros((padn,), jnp.int32)])
    colg = jnp.concatenate([col, jnp.zeros((padn,), jnp.int32)])
    colsp = jnp.concatenate([col, jnp.full((padn,), COLPAD, jnp.int32)])
    cq = jnp.concatenate([
        jnp.where((colsp >= p * NQ) & (colsp < (p + 1) * NQ),
                  colsp - p * NQ, TRASH) for p in range(QP)])
    zeros128 = jnp.zeros((ZSLAB, 128), F32)
    cq2 = cq.reshape(-1, CH)

    gather = _make_gather()
    scatter = _make_scatter()
    cntw = _make_counts()(colsp).reshape(NW, NPAD).T

    ps = params
    We1, bE1, We2, bE2 = ps["c1_e"]
    An1, aB1, An2, aB2 = ps["c1_n1"]

    x8 = jnp.pad(x, ((0, 0), (0, 6)))
    wu = _pad_rows(jnp.concatenate([We1[:2], An1[:2]], axis=1), 8)
    wv = _pad_cols(_pad_rows(We1[2:4], 8), 128)
    ts0, td0 = _pre_l1(x8, wu, wv)
    x0 = jnp.pad(x, ((0, 0), (0, 62)))
    eh0 = jnp.pad(edge_attr, ((0, padn), (64, 60)))

    we_s, w_s, be_s, wx_s, wn_s, wts_s, wtd_s, bn_s = [], [], [], [], [], [], [], []
    for li, name in enumerate(("c1", "c2", "c3", "c4")):
        We1, bE1, We2, bE2 = ps[name + "_e"]
        An1, aB1, An2, aB2 = ps[name + "_n1"]
        Bn1, bB1, Bn2, bB2 = ps[name + "_n2"]
        nxt = ("c2", "c3", "c4", "c5")[li]
        We1n = ps[nxt + "_e"][0]
        if li == 0:
            we, wA, wx, wB = _pad_rows(We1[4:8], 64), An1[2:66], \
                _pad_rows(Bn1[:2], 64), Bn1[2:66]
        else:
            we, wA, wx, wB = We1[128:192], An1[64:128], Bn1[:64], Bn1[64:128]
        if li < 3:
            An1n = ps[nxt + "_n1"][0]
            wts = jnp.concatenate([We1n[:64], An1n[:64]], axis=1)
        else:
            wts = _pad_cols(We1n[:64], 128)
        we_s.append(we)
        w_s.append(jnp.stack([We2, wA, An2]))
        be_s.append(_bstack(bE1, bE2, aB1, aB2))
        wx_s.append(wx)
        wn_s.append(jnp.stack([wB, Bn2]))
        wts_s.append(wts)
        wtd_s.append(_pad_cols(We1n[64:128], 128))
        bn_s.append(_bstack(bB1, bB2))
    ws = tuple(jnp.stack(a) for a in
               (we_s, w_s, be_s, wx_s, wn_s, wts_s, wtd_s, bn_s))

    def body(carry, lw):
        xcur, ts, td, eh = carry
        we, w, be, wx, wn, wts, wtd, bn = lw
        us, vd = gather(ts, rowp, td, colg)
        eh = _edge(us, vd, eh, we, w, be)
        part = scatter(eh, cq2, zeros128).reshape(NC, NPAD, 128)
        xcur, ts, td = _node(xcur, part, cntw, wx, wn, wts, wtd, bn)
        return (xcur, ts, td, eh), None

    (_, ts, td, eh), _ = lax.scan(body, (x0, ts0, td0, eh0), ws)

    We1, bE1, We2, bE2 = ps["c5_e"]
    us, vd = gather(ts, rowp, td, colg)
    w2 = jnp.pad(We2, ((0, 0), (0, 7)))
    b = _bstack(bE1, jnp.pad(bE2, (0, 63)))
    o8 = _edge_l5(us, vd, eh, We1[128:192], w2, b)
    return o8[:E, 0:1]


# speedup vs baseline: 1.0926x; 1.0926x over previous
import functools

import jax
import jax.numpy as jnp
from jax import lax
from jax.experimental import pallas as pl
from jax.experimental.pallas import tpu as pltpu
from jax.experimental.pallas import tpu_sc as plsc

F32 = jnp.float32

N = 10000
E = 160000
NC, NS = 2, 16
NW = NC * NS
EP = 5120
CH = 128
NCHUNK = EP // CH
PAD_E = NW * EP
NQ = 5120
QP = 2
NPAD = QP * NQ
ACC_R = 5248
ZSLAB = ACC_R // NS
FSLAB = NQ // NS
TRASH = 5240
COLPAD = 10200


def _mesh():
    return plsc.VectorSubcoreMesh(
        core_axis_name="c", subcore_axis_name="s", num_cores=NC, num_subcores=NS)


@functools.lru_cache(maxsize=None)
def _make_gather():
    @functools.partial(
        pl.kernel,
        mesh=_mesh(),
        out_type=(
            jax.ShapeDtypeStruct((PAD_E, 128), F32),
            jax.ShapeDtypeStruct((PAD_E, 128), F32),
        ),
        scratch_types=[
            pltpu.VMEM((8 * CH,), jnp.int32),
            pltpu.VMEM((8 * CH,), jnp.int32),
            pltpu.VMEM((2, CH, 128), F32),
            pltpu.VMEM((2, CH, 128), F32),
            pltpu.SemaphoreType.DMA,
            pltpu.SemaphoreType.DMA,
            pltpu.SemaphoreType.DMA,
            pltpu.SemaphoreType.DMA,
        ],
    )
    def gather(ta, ia, tb, ib, oa, ob, iva, ivb, bufa, bufb,
               sga, sgb, swa, swb):
        wid = lax.axis_index("s") * NC + lax.axis_index("c")
        base = wid * EP

        SLABC = 8
        for s in range(NCHUNK // SLABC):
            j0 = s * SLABC
            pltpu.sync_copy(ia.at[pl.ds(base + j0 * CH, SLABC * CH)], iva)
            pltpu.sync_copy(ib.at[pl.ds(base + j0 * CH, SLABC * CH)], ivb)

            @pl.loop(0, SLABC, step=2)
            def step(jj):
                for b in (0, 1):
                    lc = jj + b
                    j = j0 + lc
                    off = base + j * CH

                    @pl.when(j >= 2)
                    def _():
                        pltpu.make_async_copy(
                            bufa.at[b], oa.at[pl.ds(off, CH)], swa).wait()
                        pltpu.make_async_copy(
                            bufb.at[b], ob.at[pl.ds(off, CH)], swb).wait()

                    ca = pltpu.async_copy(
                        ta.at[iva.at[pl.ds(lc * CH, CH)]], bufa.at[b], sga)
                    cb = pltpu.async_copy(
                        tb.at[ivb.at[pl.ds(lc * CH, CH)]], bufb.at[b], sgb)
                    ca.wait()
                    cb.wait()
                    pltpu.async_copy(bufa.at[b], oa.at[pl.ds(off, CH)], swa)
                    pltpu.async_copy(bufb.at[b], ob.at[pl.ds(off, CH)], swb)

        for b in (0, 1):
            pltpu.make_async_copy(bufa.at[b], oa.at[pl.ds(base, CH)], swa).wait()
            pltpu.make_async_copy(bufb.at[b], ob.at[pl.ds(base, CH)], swb).wait()

    return gather


@functools.lru_cache(maxsize=None)
def _make_scatter():
    @functools.partial(
        pl.kernel,
        mesh=_mesh(),
        out_type=jax.ShapeDtypeStruct((NC * NPAD, 128), F32),
        scratch_types=[
            pltpu.VMEM((2, CH), jnp.int32),
            pltpu.VMEM((2, CH, 128), F32),
            pltpu.VMEM((FSLAB, 128), F32),
            pltpu.VMEM_SHARED((ACC_R, 128), F32),
            pltpu.SemaphoreType.DMA,
        ],
    )
    def scat(rows, cq2, zeros, out, iv, rbuf, stage, acc, sadd):
        cid = lax.axis_index("c")
        sid = lax.axis_index("s")
        wid = sid * NC + cid
        base = wid * EP

        for p in range(QP):
            pltpu.sync_copy(zeros, acc.at[pl.ds(sid * ZSLAB, ZSLAB)])
            plsc.subcore_barrier()

            @pl.loop(0, NCHUNK, step=2)
            def step(j0):
                for b in (0, 1):
                    j = j0 + b
                    off = base + j * CH

                    @pl.when(j >= 2)
                    def _():
                        pltpu.make_async_copy(
                            rbuf.at[b], acc.at[iv.at[b]], sadd).wait()

                    pltpu.sync_copy(
                        cq2.at[p * (PAD_E // CH) + wid * NCHUNK + j], iv.at[b])
                    pltpu.sync_copy(rows.at[pl.ds(off, CH)], rbuf.at[b])
                    pltpu.async_copy(rbuf.at[b], acc.at[iv.at[b]], sadd,
                                     add=True)

            for b in (0, 1):
                pltpu.make_async_copy(rbuf.at[b], acc.at[iv.at[b]],
                                      sadd).wait()
            plsc.subcore_barrier()
            pltpu.sync_copy(acc.at[pl.ds(sid * FSLAB, FSLAB)], stage)
            pltpu.sync_copy(
                stage,
                out.at[pl.ds(cid * NPAD + p * NQ + sid * FSLAB, FSLAB)])
            plsc.subcore_barrier()

    return scat


@functools.lru_cache(maxsize=None)
def _make_counts():
    @functools.partial(
        pl.kernel,
        mesh=_mesh(),
        compiler_params=pltpu.CompilerParams(needs_layout_passes=False),
        out_type=jax.ShapeDtypeStruct((NW * NPAD,), F32),
        scratch_types=[
            pltpu.VMEM((1024,), jnp.int32),
            pltpu.VMEM((NPAD,), F32),
        ],
    )
    def cnt(cols, out, iv, acc):
        cid = lax.axis_index("c")
        sid = lax.axis_index("s")
        wid = sid * NC + cid
        base = wid * EP
        zero = jnp.zeros((16,), F32)
        ones = jnp.ones((16,), F32)

        def zloop(i, carry):
            acc[pl.ds(i * 16, 16)] = zero
            return carry

        lax.fori_loop(0, NPAD // 16, zloop, 0)

        def body(j, carry):
            pltpu.sync_copy(cols.at[pl.ds(base + j * 1024, 1024)], iv)

            def inner(k, c2):
                plsc.addupdate_scatter(acc, [iv[pl.ds(k * 16, 16)]], ones)
                return c2

            lax.fori_loop(0, 64, inner, 0)
            return carry

        lax.fori_loop(0, EP // 1024, body, 0)
        pltpu.sync_copy(acc, out.at[pl.ds(wid * NPAD, NPAD)])

    return cnt


BE = 2048
GE = PAD_E // BE
BN = 1000
GN = N // BN


def _dot(a, b):
    return jnp.dot(a, b, preferred_element_type=F32)


def _pre_body(x_ref, wu_ref, wv_ref, u_ref, v_ref):
    x = x_ref[...]
    u_ref[...] = _dot(x, wu_ref[...])
    v_ref[...] = _dot(x, wv_ref[...])


def _pre_l1(x8, wu, wv):
    return pl.pallas_call(
        _pre_body,
        grid=(GN,),
        in_specs=[
            pl.BlockSpec((BN, 8), lambda i: (i, 0)),
            pl.BlockSpec((8, 128), lambda i: (0, 0)),
            pl.BlockSpec((8, 128), lambda i: (0, 0)),
        ],
        out_specs=[
            pl.BlockSpec((BN, 128), lambda i: (i, 0)),
            pl.BlockSpec((BN, 128), lambda i: (i, 0)),
        ],
        out_shape=[
            jax.ShapeDtypeStruct((N, 128), F32),
            jax.ShapeDtypeStruct((N, 128), F32),
        ],
    )(x8, wu, wv)


def _edge_body(us_ref, vd_ref, ea_ref, we_ref, w_ref, b_ref, eh_ref):
    u = us_ref[...]
    b = b_ref[...]
    t = (u[:, :64] + vd_ref[...][:, :64]
         + _dot(ea_ref[...][:, 64:], we_ref[...]) + b[0:1])
    t = jnp.maximum(t, 0.0)
    ea2 = _dot(t, w_ref[0]) + b[1:2]
    z = u[:, 64:] + _dot(ea2, w_ref[1]) + b[2:3]
    z = jnp.maximum(z, 0.0)
    h = _dot(z, w_ref[2]) + b[3:4]
    eh_ref[...] = jnp.concatenate([h, ea2], axis=1)


def _edge(us, vd, ea, we, w, b):
    return pl.pallas_call(
        _edge_body,
        grid=(GE,),
        in_specs=[
            pl.BlockSpec((BE, 128), lambda i: (i, 0)),
            pl.BlockSpec((BE, 128), lambda i: (i, 0)),
            pl.BlockSpec((BE, 128), lambda i: (i, 0)),
            pl.BlockSpec((64, 64), lambda i: (0, 0)),
            pl.BlockSpec((3, 64, 64), lambda i: (0, 0, 0)),
            pl.BlockSpec((8, 64), lambda i: (0, 0)),
        ],
        out_specs=pl.BlockSpec((BE, 128), lambda i: (i, 0)),
        out_shape=jax.ShapeDtypeStruct((PAD_E, 128), F32),
    )(us, vd, ea, we, w, b)


def _edge5_body(us_ref, vd_ref, ea_ref, we_ref, w2_ref, b_ref, o_ref):
    u = us_ref[...]
    b = b_ref[...]
    t = (u[:, :64] + vd_ref[...][:, :64]
         + _dot(ea_ref[...][:, 64:], we_ref[...]) + b[0:1])
    t = jnp.maximum(t, 0.0)
    o = _dot(t, w2_ref[...]) + b[1:2, :8]
    o_ref[...] = jax.nn.sigmoid(o)


def _edge_l5(us, vd, ea, we, w2, b):
    return pl.pallas_call(
        _edge5_body,
        grid=(GE,),
        in_specs=[
            pl.BlockSpec((BE, 128), lambda i: (i, 0)),
            pl.BlockSpec((BE, 128), lambda i: (i, 0)),
            pl.BlockSpec((BE, 128), lambda i: (i, 0)),
            pl.BlockSpec((64, 64), lambda i: (0, 0)),
            pl.BlockSpec((64, 8), lambda i: (0, 0)),
            pl.BlockSpec((8, 64), lambda i: (0, 0)),
        ],
        out_specs=pl.BlockSpec((BE, 8), lambda i: (i, 0)),
        out_shape=jax.ShapeDtypeStruct((PAD_E, 8), F32),
    )(us, vd, ea, we, w2, b)


def _node_body(x_ref, p_ref, cp_ref, wx_ref, w_ref, wts_ref, wtd_ref, b_ref,
               xo_ref, ts_ref, td_ref):
    p = p_ref[...]
    cp = cp_ref[...]
    b = b_ref[...]
    s = p[0, :, :64] + p[1, :, :64]
    c = jnp.sum(cp, axis=1, keepdims=True)
    agg = s / jnp.maximum(c, 1.0)
    t = _dot(x_ref[...], wx_ref[...]) + _dot(agg, w_ref[0]) + b[0:1]
    t = jnp.maximum(t, 0.0)
    xo = _dot(t, w_ref[1]) + b[1:2]
    xo_ref[...] = xo
    ts_ref[...] = _dot(xo, wts_ref[...])
    td_ref[...] = _dot(xo, wtd_ref[...])


def _node(x, part, cntp, wx, w, wts, wtd, b):
    return pl.pallas_call(
        _node_body,
        grid=(GN,),
        in_specs=[
            pl.BlockSpec((BN, 64), lambda i: (i, 0)),
            pl.BlockSpec((NC, BN, 128), lambda i: (0, i, 0)),
            pl.BlockSpec((BN, NW), lambda i: (i, 0)),
            pl.BlockSpec((64, 64), lambda i: (0, 0)),
            pl.BlockSpec((2, 64, 64), lambda i: (0, 0, 0)),
            pl.BlockSpec((64, 128), lambda i: (0, 0)),
            pl.BlockSpec((64, 128), lambda i: (0, 0)),
            pl.BlockSpec((8, 64), lambda i: (0, 0)),
        ],
        out_specs=[
            pl.BlockSpec((BN, 64), lambda i: (i, 0)),
            pl.BlockSpec((BN, 128), lambda i: (i, 0)),
            pl.BlockSpec((BN, 128), lambda i: (i, 0)),
        ],
        out_shape=[
            jax.ShapeDtypeStruct((N, 64), F32),
            jax.ShapeDtypeStruct((N, 128), F32),
            jax.ShapeDtypeStruct((N, 128), F32),
        ],
    )(x, part, cntp, wx, w, wts, wtd, b)


def _pad_rows(a, rows):
    return jnp.concatenate(
        [a, jnp.zeros((rows - a.shape[0],) + a.shape[1:], a.dtype)], axis=0)


def _pad_cols(a, cols):
    return jnp.pad(a, ((0, 0), (0, cols - a.shape[1])))


def _bstack(*bs):
    out = jnp.zeros((8, 64), F32)
    for i, b in enumerate(bs):
        out = out.at[i, : b.shape[0]].set(b)
    return out


def kernel(x, edge_index, edge_attr, params):
    row, col = edge_index[0], edge_index[1]
    padn = PAD_E - E
    rowp = jnp.concatenate([row, jnp.zeros((padn,), jnp.int32)])
    colg = jnp.concatenate([col, jnp.zeros((padn,), jnp.int32)])
    colsp = jnp.concatenate([col, jnp.full((padn,), COLPAD, jnp.int32)])
    cq = jnp.concatenate([
        jnp.where((colsp >= p * NQ) & (colsp < (p + 1) * NQ),
                  colsp - p * NQ, TRASH) for p in range(QP)])
    zeros128 = jnp.zeros((ZSLAB, 128), F32)
    cq2 = cq.reshape(-1, CH)

    gather = _make_gather()
    scatter = _make_scatter()
    cntw = _make_counts()(colsp).reshape(NW, NPAD).T

    ps = params
    We1, bE1, We2, bE2 = ps["c1_e"]
    An1, aB1, An2, aB2 = ps["c1_n1"]

    x8 = jnp.pad(x, ((0, 0), (0, 6)))
    wu = _pad_rows(jnp.concatenate([We1[:2], An1[:2]], axis=1), 8)
    wv = _pad_cols(_pad_rows(We1[2:4], 8), 128)
    ts0, td0 = _pre_l1(x8, wu, wv)
    x0 = jnp.pad(x, ((0, 0), (0, 62)))
    eh0 = jnp.pad(edge_attr, ((0, padn), (64, 60)))

    we_s, w_s, be_s, wx_s, wn_s, wts_s, wtd_s, bn_s = [], [], [], [], [], [], [], []
    for li, name in enumerate(("c1", "c2", "c3", "c4")):
        We1, bE1, We2, bE2 = ps[name + "_e"]
        An1, aB1, An2, aB2 = ps[name + "_n1"]
        Bn1, bB1, Bn2, bB2 = ps[name + "_n2"]
        nxt = ("c2", "c3", "c4", "c5")[li]
        We1n = ps[nxt + "_e"][0]
        if li == 0:
            we, wA, wx, wB = _pad_rows(We1[4:8], 64), An1[2:66], \
                _pad_rows(Bn1[:2], 64), Bn1[2:66]
        else:
            we, wA, wx, wB = We1[128:192], An1[64:128], Bn1[:64], Bn1[64:128]
        if li < 3:
            An1n = ps[nxt + "_n1"][0]
            wts = jnp.concatenate([We1n[:64], An1n[:64]], axis=1)
        else:
            wts = _pad_cols(We1n[:64], 128)
        we_s.append(we)
        w_s.append(jnp.stack([We2, wA, An2]))
        be_s.append(_bstack(bE1, bE2, aB1, aB2))
        wx_s.append(wx)
        wn_s.append(jnp.stack([wB, Bn2]))
        wts_s.append(wts)
        wtd_s.append(_pad_cols(We1n[64:128], 128))
        bn_s.append(_bstack(bB1, bB2))
    ws = tuple(jnp.stack(a) for a in
               (we_s, w_s, be_s, wx_s, wn_s, wts_s, wtd_s, bn_s))

    def body(carry, lw):
        xcur, ts, td, eh = carry
        we, w, be, wx, wn, wts, wtd, bn = lw
        us, vd = gather(ts, rowp, td, colg)
        eh = _edge(us, vd, eh, we, w, be)
        part = scatter(eh, cq2, zeros128).reshape(NC, NPAD, 128)
        xcur, ts, td = _node(xcur, part, cntw, wx, wn, wts, wtd, bn)
        return (xcur, ts, td, eh), None

    (_, ts, td, eh), _ = lax.scan(body, (x0, ts0, td0, eh0), ws)

    We1, bE1, We2, bE2 = ps["c5_e"]
    us, vd = gather(ts, rowp, td, colg)
    w2 = jnp.pad(We2, ((0, 0), (0, 7)))
    b = _bstack(bE1, jnp.pad(bE2, (0, 63)))
    o8 = _edge_l5(us, vd, eh, We1[128:192], w2, b)
    return o8[:E, 0:1]


# speedup vs baseline: 1.1323x; 1.0363x over previous
import functools

import jax
import jax.numpy as jnp
from jax import lax
from jax.experimental import pallas as pl
from jax.experimental.pallas import tpu as pltpu
from jax.experimental.pallas import tpu_sc as plsc

F32 = jnp.float32

N = 10000
E = 160000
NC, NS = 2, 16
NW = NC * NS
EP = 5120
CH = 128
NCHUNK = EP // CH
PAD_E = NW * EP
NQ = 5120
QP = 2
NPAD = QP * NQ
ACC_R = 5248
ZSLAB = ACC_R // NS
FSLAB = NQ // NS
TRASH = 5240
COLPAD = 10200


def _mesh():
    return plsc.VectorSubcoreMesh(
        core_axis_name="c", subcore_axis_name="s", num_cores=NC, num_subcores=NS)


@functools.lru_cache(maxsize=None)
def _make_gather():
    @functools.partial(
        pl.kernel,
        mesh=_mesh(),
        out_type=(
            jax.ShapeDtypeStruct((PAD_E, 128), F32),
            jax.ShapeDtypeStruct((PAD_E, 128), F32),
        ),
        scratch_types=[
            pltpu.VMEM((8 * CH,), jnp.int32),
            pltpu.VMEM((8 * CH,), jnp.int32),
            pltpu.VMEM((2, CH, 128), F32),
            pltpu.VMEM((2, CH, 128), F32),
            pltpu.SemaphoreType.DMA,
            pltpu.SemaphoreType.DMA,
            pltpu.SemaphoreType.DMA,
            pltpu.SemaphoreType.DMA,
        ],
    )
    def gather(ta, ia, tb, ib, oa, ob, iva, ivb, bufa, bufb,
               sga, sgb, swa, swb):
        wid = lax.axis_index("s") * NC + lax.axis_index("c")
        base = wid * EP

        SLABC = 8
        for s in range(NCHUNK // SLABC):
            j0 = s * SLABC
            pltpu.sync_copy(ia.at[pl.ds(base + j0 * CH, SLABC * CH)], iva)
            pltpu.sync_copy(ib.at[pl.ds(base + j0 * CH, SLABC * CH)], ivb)

            @pl.loop(0, SLABC, step=2)
            def step(jj):
                for b in (0, 1):
                    lc = jj + b
                    j = j0 + lc
                    off = base + j * CH

                    @pl.when(j >= 2)
                    def _():
                        pltpu.make_async_copy(
                            bufa.at[b], oa.at[pl.ds(off, CH)], swa).wait()
                        pltpu.make_async_copy(
                            bufb.at[b], ob.at[pl.ds(off, CH)], swb).wait()

                    ca = pltpu.async_copy(
                        ta.at[iva.at[pl.ds(lc * CH, CH)]], bufa.at[b], sga)
                    cb = pltpu.async_copy(
                        tb.at[ivb.at[pl.ds(lc * CH, CH)]], bufb.at[b], sgb)
                    ca.wait()
                    cb.wait()
                    pltpu.async_copy(bufa.at[b], oa.at[pl.ds(off, CH)], swa)
                    pltpu.async_copy(bufb.at[b], ob.at[pl.ds(off, CH)], swb)

        for b in (0, 1):
            pltpu.make_async_copy(bufa.at[b], oa.at[pl.ds(base, CH)], swa).wait()
            pltpu.make_async_copy(bufb.at[b], ob.at[pl.ds(base, CH)], swb).wait()

    return gather


@functools.lru_cache(maxsize=None)
def _make_scatter():
    @functools.partial(
        pl.kernel,
        mesh=_mesh(),
        out_type=jax.ShapeDtypeStruct((NC * NPAD, 128), F32),
        scratch_types=[
            pltpu.VMEM((2, CH), jnp.int32),
            pltpu.VMEM((2, CH, 128), F32),
            pltpu.VMEM((FSLAB, 128), F32),
            pltpu.VMEM_SHARED((ACC_R, 128), F32),
            pltpu.SemaphoreType.DMA,
        ],
    )
    def scat(rows, cq2, zeros, out, iv, rbuf, stage, acc, sadd):
        cid = lax.axis_index("c")
        sid = lax.axis_index("s")
        wid = sid * NC + cid
        base = wid * EP

        for p in range(QP):
            pltpu.sync_copy(zeros, acc.at[pl.ds(sid * ZSLAB, ZSLAB)])
            plsc.subcore_barrier()

            @pl.loop(0, NCHUNK, step=2)
            def step(j0):
                for b in (0, 1):
                    j = j0 + b
                    off = base + j * CH

                    @pl.when(j >= 2)
                    def _():
                        pltpu.make_async_copy(
                            rbuf.at[b], acc.at[iv.at[b]], sadd).wait()

                    pltpu.sync_copy(
                        cq2.at[p * (PAD_E // CH) + wid * NCHUNK + j], iv.at[b])
                    pltpu.sync_copy(rows.at[pl.ds(off, CH)], rbuf.at[b])
                    pltpu.async_copy(rbuf.at[b], acc.at[iv.at[b]], sadd,
                                     add=True)

            for b in (0, 1):
                pltpu.make_async_copy(rbuf.at[b], acc.at[iv.at[b]],
                                      sadd).wait()
            plsc.subcore_barrier()
            pltpu.sync_copy(acc.at[pl.ds(sid * FSLAB, FSLAB)], stage)
            pltpu.sync_copy(
                stage,
                out.at[pl.ds(cid * NPAD + p * NQ + sid * FSLAB, FSLAB)])
            plsc.subcore_barrier()

    return scat


@functools.lru_cache(maxsize=None)
def _make_counts():
    @functools.partial(
        pl.kernel,
        mesh=_mesh(),
        compiler_params=pltpu.CompilerParams(needs_layout_passes=False),
        out_type=jax.ShapeDtypeStruct((NW * NPAD,), F32),
        scratch_types=[
            pltpu.VMEM((1024,), jnp.int32),
            pltpu.VMEM((NPAD,), F32),
        ],
    )
    def cnt(cols, out, iv, acc):
        cid = lax.axis_index("c")
        sid = lax.axis_index("s")
        wid = sid * NC + cid
        base = wid * EP
        zero = jnp.zeros((16,), F32)
        ones = jnp.ones((16,), F32)

        def zloop(i, carry):
            acc[pl.ds(i * 16, 16)] = zero
            return carry

        lax.fori_loop(0, NPAD // 16, zloop, 0)

        def body(j, carry):
            pltpu.sync_copy(cols.at[pl.ds(base + j * 1024, 1024)], iv)

            def inner(k, c2):
                plsc.addupdate_scatter(acc, [iv[pl.ds(k * 16, 16)]], ones)
                return c2

            lax.fori_loop(0, 64, inner, 0)
            return carry

        lax.fori_loop(0, EP // 1024, body, 0)
        pltpu.sync_copy(acc, out.at[pl.ds(wid * NPAD, NPAD)])

    return cnt


BE = 4096
GE = PAD_E // BE
BN = 1000
GN = N // BN


def _dot(a, b):
    return jnp.dot(a, b, preferred_element_type=F32)


def _pre_body(x_ref, wu_ref, wv_ref, u_ref, v_ref):
    x = x_ref[...]
    u_ref[...] = _dot(x, wu_ref[...])
    v_ref[...] = _dot(x, wv_ref[...])


def _pre_l1(x8, wu, wv):
    return pl.pallas_call(
        _pre_body,
        grid=(GN,),
        in_specs=[
            pl.BlockSpec((BN, 8), lambda i: (i, 0)),
            pl.BlockSpec((8, 128), lambda i: (0, 0)),
            pl.BlockSpec((8, 128), lambda i: (0, 0)),
        ],
        out_specs=[
            pl.BlockSpec((BN, 128), lambda i: (i, 0)),
            pl.BlockSpec((BN, 128), lambda i: (i, 0)),
        ],
        out_shape=[
            jax.ShapeDtypeStruct((N, 128), F32),
            jax.ShapeDtypeStruct((N, 128), F32),
        ],
    )(x8, wu, wv)


def _edge_body(us_ref, vd_ref, ea_ref, we_ref, w_ref, b_ref, eh_ref):
    u = us_ref[...]
    b = b_ref[...]
    t = (u[:, :64] + vd_ref[...][:, :64]
         + _dot(ea_ref[...][:, 64:], we_ref[...]) + b[0:1])
    t = jnp.maximum(t, 0.0)
    ea2 = _dot(t, w_ref[0]) + b[1:2]
    z = u[:, 64:] + _dot(ea2, w_ref[1]) + b[2:3]
    z = jnp.maximum(z, 0.0)
    h = _dot(z, w_ref[2]) + b[3:4]
    eh_ref[...] = jnp.concatenate([h, ea2], axis=1)


def _edge(us, vd, ea, we, w, b):
    return pl.pallas_call(
        _edge_body,
        grid=(GE,),
        in_specs=[
            pl.BlockSpec((BE, 128), lambda i: (i, 0)),
            pl.BlockSpec((BE, 128), lambda i: (i, 0)),
            pl.BlockSpec((BE, 128), lambda i: (i, 0)),
            pl.BlockSpec((64, 64), lambda i: (0, 0)),
            pl.BlockSpec((3, 64, 64), lambda i: (0, 0, 0)),
            pl.BlockSpec((8, 64), lambda i: (0, 0)),
        ],
        out_specs=pl.BlockSpec((BE, 128), lambda i: (i, 0)),
        out_shape=jax.ShapeDtypeStruct((PAD_E, 128), F32),
    )(us, vd, ea, we, w, b)


def _edge5_body(us_ref, vd_ref, ea_ref, we_ref, w2_ref, b_ref, o_ref):
    u = us_ref[...]
    b = b_ref[...]
    t = (u[:, :64] + vd_ref[...][:, :64]
         + _dot(ea_ref[...][:, 64:], we_ref[...]) + b[0:1])
    t = jnp.maximum(t, 0.0)
    o = _dot(t, w2_ref[...]) + b[1:2, :8]
    o_ref[...] = jax.nn.sigmoid(o)


def _edge_l5(us, vd, ea, we, w2, b):
    return pl.pallas_call(
        _edge5_body,
        grid=(GE,),
        in_specs=[
            pl.BlockSpec((BE, 128), lambda i: (i, 0)),
            pl.BlockSpec((BE, 128), lambda i: (i, 0)),
            pl.BlockSpec((BE, 128), lambda i: (i, 0)),
            pl.BlockSpec((64, 64), lambda i: (0, 0)),
            pl.BlockSpec((64, 8), lambda i: (0, 0)),
            pl.BlockSpec((8, 64), lambda i: (0, 0)),
        ],
        out_specs=pl.BlockSpec((BE, 8), lambda i: (i, 0)),
        out_shape=jax.ShapeDtypeStruct((PAD_E, 8), F32),
    )(us, vd, ea, we, w2, b)


def _node_body(x_ref, p_ref, cp_ref, wx_ref, w_ref, wts_ref, wtd_ref, b_ref,
               xo_ref, ts_ref, td_ref):
    p = p_ref[...]
    cp = cp_ref[...]
    b = b_ref[...]
    s = p[0, :, :64] + p[1, :, :64]
    c = jnp.sum(cp, axis=1, keepdims=True)
    agg = s / jnp.maximum(c, 1.0)
    t = _dot(x_ref[...], wx_ref[...]) + _dot(agg, w_ref[0]) + b[0:1]
    t = jnp.maximum(t, 0.0)
    xo = _dot(t, w_ref[1]) + b[1:2]
    xo_ref[...] = xo
    ts_ref[...] = _dot(xo, wts_ref[...])
    td_ref[...] = _dot(xo, wtd_ref[...])


def _node(x, part, cntp, wx, w, wts, wtd, b):
    return pl.pallas_call(
        _node_body,
        grid=(GN,),
        in_specs=[
            pl.BlockSpec((BN, 64), lambda i: (i, 0)),
            pl.BlockSpec((NC, BN, 128), lambda i: (0, i, 0)),
            pl.BlockSpec((BN, NW), lambda i: (i, 0)),
            pl.BlockSpec((64, 64), lambda i: (0, 0)),
            pl.BlockSpec((2, 64, 64), lambda i: (0, 0, 0)),
            pl.BlockSpec((64, 128), lambda i: (0, 0)),
            pl.BlockSpec((64, 128), lambda i: (0, 0)),
            pl.BlockSpec((8, 64), lambda i: (0, 0)),
        ],
        out_specs=[
            pl.BlockSpec((BN, 64), lambda i: (i, 0)),
            pl.BlockSpec((BN, 128), lambda i: (i, 0)),
            pl.BlockSpec((BN, 128), lambda i: (i, 0)),
        ],
        out_shape=[
            jax.ShapeDtypeStruct((N, 64), F32),
            jax.ShapeDtypeStruct((N, 128), F32),
            jax.ShapeDtypeStruct((N, 128), F32),
        ],
    )(x, part, cntp, wx, w, wts, wtd, b)


def _pad_rows(a, rows):
    return jnp.concatenate(
        [a, jnp.zeros((rows - a.shape[0],) + a.shape[1:], a.dtype)], axis=0)


def _pad_cols(a, cols):
    return jnp.pad(a, ((0, 0), (0, cols - a.shape[1])))


def _bstack(*bs):
    out = jnp.zeros((8, 64), F32)
    for i, b in enumerate(bs):
        out = out.at[i, : b.shape[0]].set(b)
    return out


def kernel(x, edge_index, edge_attr, params):
    row, col = edge_index[0], edge_index[1]
    padn = PAD_E - E
    rowp = jnp.concatenate([row, jnp.zeros((padn,), jnp.int32)])
    colg = jnp.concatenate([col, jnp.zeros((padn,), jnp.int32)])
    colsp = jnp.concatenate([col, jnp.full((padn,), COLPAD, jnp.int32)])
    cq = jnp.concatenate([
        jnp.where((colsp >= p * NQ) & (colsp < (p + 1) * NQ),
                  colsp - p * NQ, TRASH) for p in range(QP)])
    zeros128 = jnp.zeros((ZSLAB, 128), F32)
    cq2 = cq.reshape(-1, CH)

    gather = _make_gather()
    scatter = _make_scatter()
    cntw = _make_counts()(colsp).reshape(NW, NPAD).T

    ps = params
    We1, bE1, We2, bE2 = ps["c1_e"]
    An1, aB1, An2, aB2 = ps["c1_n1"]

    x8 = jnp.pad(x, ((0, 0), (0, 6)))
    wu = _pad_rows(jnp.concatenate([We1[:2], An1[:2]], axis=1), 8)
    wv = _pad_cols(_pad_rows(We1[2:4], 8), 128)
    ts0, td0 = _pre_l1(x8, wu, wv)
    x0 = jnp.pad(x, ((0, 0), (0, 62)))
    eh0 = jnp.pad(edge_attr, ((0, padn), (64, 60)))

    we_s, w_s, be_s, wx_s, wn_s, wts_s, wtd_s, bn_s = [], [], [], [], [], [], [], []
    for li, name in enumerate(("c1", "c2", "c3", "c4")):
        We1, bE1, We2, bE2 = ps[name + "_e"]
        An1, aB1, An2, aB2 = ps[name + "_n1"]
        Bn1, bB1, Bn2, bB2 = ps[name + "_n2"]
        nxt = ("c2", "c3", "c4", "c5")[li]
        We1n = ps[nxt + "_e"][0]
        if li == 0:
            we, wA, wx, wB = _pad_rows(We1[4:8], 64), An1[2:66], \
                _pad_rows(Bn1[:2], 64), Bn1[2:66]
        else:
            we, wA, wx, wB = We1[128:192], An1[64:128], Bn1[:64], Bn1[64:128]
        if li < 3:
            An1n = ps[nxt + "_n1"][0]
            wts = jnp.concatenate([We1n[:64], An1n[:64]], axis=1)
        else:
            wts = _pad_cols(We1n[:64], 128)
        we_s.append(we)
        w_s.append(jnp.stack([We2, wA, An2]))
        be_s.append(_bstack(bE1, bE2, aB1, aB2))
        wx_s.append(wx)
        wn_s.append(jnp.stack([wB, Bn2]))
        wts_s.append(wts)
        wtd_s.append(_pad_cols(We1n[64:128], 128))
        bn_s.append(_bstack(bB1, bB2))
    ws = tuple(jnp.stack(a) for a in
               (we_s, w_s, be_s, wx_s, wn_s, wts_s, wtd_s, bn_s))

    def body(carry, lw):
        xcur, ts, td, eh = carry
        we, w, be, wx, wn, wts, wtd, bn = lw
        us, vd = gather(ts, rowp, td, colg)
        eh = _edge(us, vd, eh, we, w, be)
        part = scatter(eh, cq2, zeros128).reshape(NC, NPAD, 128)
        xcur, ts, td = _node(xcur, part, cntw, wx, wn, wts, wtd, bn)
        return (xcur, ts, td, eh), None

    (_, ts, td, eh), _ = lax.scan(body, (x0, ts0, td0, eh0), ws)

    We1, bE1, We2, bE2 = ps["c5_e"]
    us, vd = gather(ts, rowp, td, colg)
    w2 = jnp.pad(We2, ((0, 0), (0, 7)))
    b = _bstack(bE1, jnp.pad(bE2, (0, 63)))
    o8 = _edge_l5(us, vd, eh, We1[128:192], w2, b)
    return o8[:E, 0:1]


# speedup vs baseline: 1.1466x; 1.0126x over previous
import functools

import jax
import jax.numpy as jnp
from jax import lax
from jax.experimental import pallas as pl
from jax.experimental.pallas import tpu as pltpu
from jax.experimental.pallas import tpu_sc as plsc

F32 = jnp.float32

N = 10000
E = 160000
NC, NS = 2, 16
NW = NC * NS
EP = 5120
CH = 128
NCHUNK = EP // CH
PAD_E = NW * EP
NQ = 5120
QP = 2
NPAD = QP * NQ
ACC_R = 5248
ZSLAB = ACC_R // NS
FSLAB = NQ // NS
TRASH = 5240
COLPAD = 10200


def _mesh():
    return plsc.VectorSubcoreMesh(
        core_axis_name="c", subcore_axis_name="s", num_cores=NC, num_subcores=NS)


@functools.lru_cache(maxsize=None)
def _make_gather():
    @functools.partial(
        pl.kernel,
        mesh=_mesh(),
        out_type=(
            jax.ShapeDtypeStruct((PAD_E, 128), F32),
            jax.ShapeDtypeStruct((PAD_E, 128), F32),
        ),
        scratch_types=[
            pltpu.VMEM((8 * CH,), jnp.int32),
            pltpu.VMEM((8 * CH,), jnp.int32),
            pltpu.VMEM((2, CH, 128), F32),
            pltpu.VMEM((2, CH, 128), F32),
            pltpu.SemaphoreType.DMA,
            pltpu.SemaphoreType.DMA,
            pltpu.SemaphoreType.DMA,
            pltpu.SemaphoreType.DMA,
        ],
    )
    def gather(ta, ia, tb, ib, oa, ob, iva, ivb, bufa, bufb,
               sga, sgb, swa, swb):
        wid = lax.axis_index("s") * NC + lax.axis_index("c")
        base = wid * EP

        SLABC = 8
        for s in range(NCHUNK // SLABC):
            j0 = s * SLABC
            pltpu.sync_copy(ia.at[pl.ds(base + j0 * CH, SLABC * CH)], iva)
            pltpu.sync_copy(ib.at[pl.ds(base + j0 * CH, SLABC * CH)], ivb)

            @pl.loop(0, SLABC, step=2)
            def step(jj):
                for b in (0, 1):
                    lc = jj + b
                    j = j0 + lc
                    off = base + j * CH

                    @pl.when(j >= 2)
                    def _():
                        pltpu.make_async_copy(
                            bufa.at[b], oa.at[pl.ds(off, CH)], swa).wait()
                        pltpu.make_async_copy(
                            bufb.at[b], ob.at[pl.ds(off, CH)], swb).wait()

                    ca = pltpu.async_copy(
                        ta.at[iva.at[pl.ds(lc * CH, CH)]], bufa.at[b], sga)
                    cb = pltpu.async_copy(
                        tb.at[ivb.at[pl.ds(lc * CH, CH)]], bufb.at[b], sgb)
                    ca.wait()
                    cb.wait()
                    pltpu.async_copy(bufa.at[b], oa.at[pl.ds(off, CH)], swa)
                    pltpu.async_copy(bufb.at[b], ob.at[pl.ds(off, CH)], swb)

        for b in (0, 1):
            pltpu.make_async_copy(bufa.at[b], oa.at[pl.ds(base, CH)], swa).wait()
            pltpu.make_async_copy(bufb.at[b], ob.at[pl.ds(base, CH)], swb).wait()

    return gather


@functools.lru_cache(maxsize=None)
def _make_scatter():
    @functools.partial(
        pl.kernel,
        mesh=_mesh(),
        out_type=jax.ShapeDtypeStruct((NC * NPAD, 128), F32),
        scratch_types=[
            pltpu.VMEM((2, CH), jnp.int32),
            pltpu.VMEM((2, CH, 128), F32),
            pltpu.VMEM((FSLAB, 128), F32),
            pltpu.VMEM_SHARED((ACC_R, 128), F32),
            pltpu.SemaphoreType.DMA,
        ],
    )
    def scat(rows, cq2, zeros, out, iv, rbuf, stage, acc, sadd):
        cid = lax.axis_index("c")
        sid = lax.axis_index("s")
        wid = sid * NC + cid
        base = wid * EP

        for p in range(QP):
            pltpu.sync_copy(zeros, acc.at[pl.ds(sid * ZSLAB, ZSLAB)])
            plsc.subcore_barrier()

            @pl.loop(0, NCHUNK, step=2)
            def step(j0):
                for b in (0, 1):
                    j = j0 + b
                    off = base + j * CH

                    @pl.when(j >= 2)
                    def _():
                        pltpu.make_async_copy(
                            rbuf.at[b], acc.at[iv.at[b]], sadd).wait()

                    pltpu.sync_copy(
                        cq2.at[p * (PAD_E // CH) + wid * NCHUNK + j], iv.at[b])
                    pltpu.sync_copy(rows.at[pl.ds(off, CH)], rbuf.at[b])
                    pltpu.async_copy(rbuf.at[b], acc.at[iv.at[b]], sadd,
                                     add=True)

            for b in (0, 1):
                pltpu.make_async_copy(rbuf.at[b], acc.at[iv.at[b]],
                                      sadd).wait()
            plsc.subcore_barrier()
            pltpu.sync_copy(acc.at[pl.ds(sid * FSLAB, FSLAB)], stage)
            pltpu.sync_copy(
                stage,
                out.at[pl.ds(cid * NPAD + p * NQ + sid * FSLAB, FSLAB)])
            plsc.subcore_barrier()

    return scat


@functools.lru_cache(maxsize=None)
def _make_counts():
    @functools.partial(
        pl.kernel,
        mesh=_mesh(),
        compiler_params=pltpu.CompilerParams(needs_layout_passes=False),
        out_type=jax.ShapeDtypeStruct((NW * NPAD,), F32),
        scratch_types=[
            pltpu.VMEM((1024,), jnp.int32),
            pltpu.VMEM((NPAD,), F32),
        ],
    )
    def cnt(cols, out, iv, acc):
        cid = lax.axis_index("c")
        sid = lax.axis_index("s")
        wid = sid * NC + cid
        base = wid * EP
        zero = jnp.zeros((16,), F32)
        ones = jnp.ones((16,), F32)

        def zloop(i, carry):
            acc[pl.ds(i * 16, 16)] = zero
            return carry

        lax.fori_loop(0, NPAD // 16, zloop, 0)

        def body(j, carry):
            pltpu.sync_copy(cols.at[pl.ds(base + j * 1024, 1024)], iv)

            def inner(k, c2):
                plsc.addupdate_scatter(acc, [iv[pl.ds(k * 16, 16)]], ones)
                return c2

            lax.fori_loop(0, 64, inner, 0)
            return carry

        lax.fori_loop(0, EP // 1024, body, 0)
        pltpu.sync_copy(acc, out.at[pl.ds(wid * NPAD, NPAD)])

    return cnt


BE = 8192
GE = PAD_E // BE
BN = 1000
GN = N // BN


def _dot(a, b):
    return jnp.dot(a, b, preferred_element_type=F32)


def _pre_body(x_ref, wu_ref, wv_ref, u_ref, v_ref):
    x = x_ref[...]
    u_ref[...] = _dot(x, wu_ref[...])
    v_ref[...] = _dot(x, wv_ref[...])


def _pre_l1(x8, wu, wv):
    return pl.pallas_call(
        _pre_body,
        grid=(GN,),
        in_specs=[
            pl.BlockSpec((BN, 8), lambda i: (i, 0)),
            pl.BlockSpec((8, 128), lambda i: (0, 0)),
            pl.BlockSpec((8, 128), lambda i: (0, 0)),
        ],
        out_specs=[
            pl.BlockSpec((BN, 128), lambda i: (i, 0)),
            pl.BlockSpec((BN, 128), lambda i: (i, 0)),
        ],
        out_shape=[
            jax.ShapeDtypeStruct((N, 128), F32),
            jax.ShapeDtypeStruct((N, 128), F32),
        ],
    )(x8, wu, wv)


def _edge_body(us_ref, vd_ref, ea_ref, we_ref, w_ref, b_ref, eh_ref):
    u = us_ref[...]
    b = b_ref[...]
    t = (u[:, :64] + vd_ref[...][:, :64]
         + _dot(ea_ref[...][:, 64:], we_ref[...]) + b[0:1])
    t = jnp.maximum(t, 0.0)
    ea2 = _dot(t, w_ref[0]) + b[1:2]
    z = u[:, 64:] + _dot(ea2, w_ref[1]) + b[2:3]
    z = jnp.maximum(z, 0.0)
    h = _dot(z, w_ref[2]) + b[3:4]
    eh_ref[...] = jnp.concatenate([h, ea2], axis=1)


def _edge(us, vd, ea, we, w, b):
    return pl.pallas_call(
        _edge_body,
        grid=(GE,),
        in_specs=[
            pl.BlockSpec((BE, 128), lambda i: (i, 0)),
            pl.BlockSpec((BE, 128), lambda i: (i, 0)),
            pl.BlockSpec((BE, 128), lambda i: (i, 0)),
            pl.BlockSpec((64, 64), lambda i: (0, 0)),
            pl.BlockSpec((3, 64, 64), lambda i: (0, 0, 0)),
            pl.BlockSpec((8, 64), lambda i: (0, 0)),
        ],
        out_specs=pl.BlockSpec((BE, 128), lambda i: (i, 0)),
        out_shape=jax.ShapeDtypeStruct((PAD_E, 128), F32),
    )(us, vd, ea, we, w, b)


def _edge5_body(us_ref, vd_ref, ea_ref, we_ref, w2_ref, b_ref, o_ref):
    u = us_ref[...]
    b = b_ref[...]
    t = (u[:, :64] + vd_ref[...][:, :64]
         + _dot(ea_ref[...][:, 64:], we_ref[...]) + b[0:1])
    t = jnp.maximum(t, 0.0)
    o = _dot(t, w2_ref[...]) + b[1:2, :8]
    o_ref[...] = jax.nn.sigmoid(o)


def _edge_l5(us, vd, ea, we, w2, b):
    return pl.pallas_call(
        _edge5_body,
        grid=(GE,),
        in_specs=[
            pl.BlockSpec((BE, 128), lambda i: (i, 0)),
            pl.BlockSpec((BE, 128), lambda i: (i, 0)),
            pl.BlockSpec((BE, 128), lambda i: (i, 0)),
            pl.BlockSpec((64, 64), lambda i: (0, 0)),
            pl.BlockSpec((64, 8), lambda i: (0, 0)),
            pl.BlockSpec((8, 64), lambda i: (0, 0)),
        ],
        out_specs=pl.BlockSpec((BE, 8), lambda i: (i, 0)),
        out_shape=jax.ShapeDtypeStruct((PAD_E, 8), F32),
    )(us, vd, ea, we, w2, b)


def _node_body(x_ref, p_ref, cp_ref, wx_ref, w_ref, wts_ref, wtd_ref, b_ref,
               xo_ref, ts_ref, td_ref):
    p = p_ref[...]
    cp = cp_ref[...]
    b = b_ref[...]
    s = p[0, :, :64] + p[1, :, :64]
    c = jnp.sum(cp, axis=1, keepdims=True)
    agg = s / jnp.maximum(c, 1.0)
    t = _dot(x_ref[...], wx_ref[...]) + _dot(agg, w_ref[0]) + b[0:1]
    t = jnp.maximum(t, 0.0)
    xo = _dot(t, w_ref[1]) + b[1:2]
    xo_ref[...] = xo
    ts_ref[...] = _dot(xo, wts_ref[...])
    td_ref[...] = _dot(xo, wtd_ref[...])


def _node(x, part, cntp, wx, w, wts, wtd, b):
    return pl.pallas_call(
        _node_body,
        grid=(GN,),
        in_specs=[
            pl.BlockSpec((BN, 64), lambda i: (i, 0)),
            pl.BlockSpec((NC, BN, 128), lambda i: (0, i, 0)),
            pl.BlockSpec((BN, NW), lambda i: (i, 0)),
            pl.BlockSpec((64, 64), lambda i: (0, 0)),
            pl.BlockSpec((2, 64, 64), lambda i: (0, 0, 0)),
            pl.BlockSpec((64, 128), lambda i: (0, 0)),
            pl.BlockSpec((64, 128), lambda i: (0, 0)),
            pl.BlockSpec((8, 64), lambda i: (0, 0)),
        ],
        out_specs=[
            pl.BlockSpec((BN, 64), lambda i: (i, 0)),
            pl.BlockSpec((BN, 128), lambda i: (i, 0)),
            pl.BlockSpec((BN, 128), lambda i: (i, 0)),
        ],
        out_shape=[
            jax.ShapeDtypeStruct((N, 64), F32),
            jax.ShapeDtypeStruct((N, 128), F32),
            jax.ShapeDtypeStruct((N, 128), F32),
        ],
    )(x, part, cntp, wx, w, wts, wtd, b)


def _pad_rows(a, rows):
    return jnp.concatenate(
        [a, jnp.zeros((rows - a.shape[0],) + a.shape[1:], a.dtype)], axis=0)


def _pad_cols(a, cols):
    return jnp.pad(a, ((0, 0), (0, cols - a.shape[1])))


def _bstack(*bs):
    out = jnp.zeros((8, 64), F32)
    for i, b in enumerate(bs):
        out = out.at[i, : b.shape[0]].set(b)
    return out


def kernel(x, edge_index, edge_attr, params):
    row, col = edge_index[0], edge_index[1]
    padn = PAD_E - E
    rowp = jnp.concatenate([row, jnp.zeros((padn,), jnp.int32)])
    colg = jnp.concatenate([col, jnp.zeros((padn,), jnp.int32)])
    colsp = jnp.concatenate([col, jnp.full((padn,), COLPAD, jnp.int32)])
    cq = jnp.concatenate([
        jnp.where((colsp >= p * NQ) & (colsp < (p + 1) * NQ),
                  colsp - p * NQ, TRASH) for p in range(QP)])
    zeros128 = jnp.zeros((ZSLAB, 128), F32)
    cq2 = cq.reshape(-1, CH)

    gather = _make_gather()
    scatter = _make_scatter()
    cntw = _make_counts()(colsp).reshape(NW, NPAD).T

    ps = params
    We1, bE1, We2, bE2 = ps["c1_e"]
    An1, aB1, An2, aB2 = ps["c1_n1"]

    x8 = jnp.pad(x, ((0, 0), (0, 6)))
    wu = _pad_rows(jnp.concatenate([We1[:2], An1[:2]], axis=1), 8)
    wv = _pad_cols(_pad_rows(We1[2:4], 8), 128)
    ts0, td0 = _pre_l1(x8, wu, wv)
    x0 = jnp.pad(x, ((0, 0), (0, 62)))
    eh0 = jnp.pad(edge_attr, ((0, padn), (64, 60)))

    we_s, w_s, be_s, wx_s, wn_s, wts_s, wtd_s, bn_s = [], [], [], [], [], [], [], []
    for li, name in enumerate(("c1", "c2", "c3", "c4")):
        We1, bE1, We2, bE2 = ps[name + "_e"]
        An1, aB1, An2, aB2 = ps[name + "_n1"]
        Bn1, bB1, Bn2, bB2 = ps[name + "_n2"]
        nxt = ("c2", "c3", "c4", "c5")[li]
        We1n = ps[nxt + "_e"][0]
        if li == 0:
            we, wA, wx, wB = _pad_rows(We1[4:8], 64), An1[2:66], \
                _pad_rows(Bn1[:2], 64), Bn1[2:66]
        else:
            we, wA, wx, wB = We1[128:192], An1[64:128], Bn1[:64], Bn1[64:128]
        if li < 3:
            An1n = ps[nxt + "_n1"][0]
            wts = jnp.concatenate([We1n[:64], An1n[:64]], axis=1)
        else:
            wts = _pad_cols(We1n[:64], 128)
        we_s.append(we)
        w_s.append(jnp.stack([We2, wA, An2]))
        be_s.append(_bstack(bE1, bE2, aB1, aB2))
        wx_s.append(wx)
        wn_s.append(jnp.stack([wB, Bn2]))
        wts_s.append(wts)
        wtd_s.append(_pad_cols(We1n[64:128], 128))
        bn_s.append(_bstack(bB1, bB2))
    ws = tuple(jnp.stack(a) for a in
               (we_s, w_s, be_s, wx_s, wn_s, wts_s, wtd_s, bn_s))

    def body(carry, lw):
        xcur, ts, td, eh = carry
        we, w, be, wx, wn, wts, wtd, bn = lw
        us, vd = gather(ts, rowp, td, colg)
        eh = _edge(us, vd, eh, we, w, be)
        part = scatter(eh, cq2, zeros128).reshape(NC, NPAD, 128)
        xcur, ts, td = _node(xcur, part, cntw, wx, wn, wts, wtd, bn)
        return (xcur, ts, td, eh), None

    (_, ts, td, eh), _ = lax.scan(body, (x0, ts0, td0, eh0), ws)

    We1, bE1, We2, bE2 = ps["c5_e"]
    us, vd = gather(ts, rowp, td, colg)
    w2 = jnp.pad(We2, ((0, 0), (0, 7)))
    b = _bstack(bE1, jnp.pad(bE2, (0, 63)))
    o8 = _edge_l5(us, vd, eh, We1[128:192], w2, b)
    return o8[:E, 0:1]
